# baseline (speedup 1.0000x reference)
import functools

import jax
import jax.numpy as jnp
from jax import lax
from jax.experimental import pallas as pl
from jax.experimental.pallas import tpu as pltpu
from jax.experimental.pallas import tpu_sc as plsc

B = 16384
D = 64
NC = 2
NS = 16
NW = NC * NS
N_PER_W = B // NW
CHUNK = 256
N_CHUNKS = N_PER_W // CHUNK
LANES = 16


def _sc_body(h_hbm, r_hbm, t_hbm, rel_hbm, table_hbm, out_hbm,
             idx_v, rows_v, h_v, r_v, t_v, out_v, sem):
    wid = lax.axis_index("s") * NC + lax.axis_index("c")
    base = wid * N_PER_W
    for c in range(N_CHUNKS):
        row0 = base + c * CHUNK
        pltpu.sync_copy(rel_hbm.at[pl.ds(row0, CHUNK)], idx_v)
        gather = pltpu.async_copy(table_hbm.at[idx_v], rows_v, sem)
        pltpu.sync_copy(h_hbm.at[pl.ds(row0, CHUNK), :], h_v)
        pltpu.sync_copy(r_hbm.at[pl.ds(row0, CHUNK), :], r_v)
        pltpu.sync_copy(t_hbm.at[pl.ds(row0, CHUNK), :], t_v)
        gather.wait()

        def group_body(g, carry, c=c):
            rowvec = lax.iota(jnp.int32, LANES) + g * LANES
            acc = jnp.zeros((LANES,), jnp.float32)
            for d in range(D):
                dvec = jnp.full((LANES,), d, jnp.int32)
                hv = plsc.load_gather(h_v, [rowvec, dvec])
                tv = plsc.load_gather(t_v, [rowvec, dvec])
                rv = plsc.load_gather(r_v, [rowvec, dvec])
                gv = plsc.load_gather(rows_v, [rowvec, dvec])
                acc = acc + hv * tv * (rv + gv)
            out_v[pl.ds(c * CHUNK + g * LANES, LANES)] = acc * 0.5
            return carry

        lax.fori_loop(0, CHUNK // LANES, group_body, 0)
    pltpu.sync_copy(out_v, out_hbm.at[pl.ds(base, N_PER_W)])


@functools.partial(
    pl.kernel,
    out_type=jax.ShapeDtypeStruct((B,), jnp.float32),
    mesh=plsc.VectorSubcoreMesh(core_axis_name="c", subcore_axis_name="s"),
    compiler_params=pltpu.CompilerParams(
        needs_layout_passes=False, use_tc_tiling_on_sc=False),
    scratch_types=[
        pltpu.VMEM((CHUNK,), jnp.int32),
        pltpu.VMEM((CHUNK, D), jnp.float32),
        pltpu.VMEM((CHUNK, D), jnp.float32),
        pltpu.VMEM((CHUNK, D), jnp.float32),
        pltpu.VMEM((CHUNK, D), jnp.float32),
        pltpu.VMEM((N_PER_W,), jnp.float32),
        pltpu.SemaphoreType.DMA,
    ],
)
def _simple_cal_sc(h_hbm, r_hbm, t_hbm, rel_hbm, table_hbm, out_hbm,
                   idx_v, rows_v, h_v, r_v, t_v, out_v, sem):
    _sc_body(h_hbm, r_hbm, t_hbm, rel_hbm, table_hbm, out_hbm,
             idx_v, rows_v, h_v, r_v, t_v, out_v, sem)


def kernel(x0, x1, x2, rel, rel_inv_table):
    h = x0.reshape(B, D)
    r = x1.reshape(B, D)
    t = x2.reshape(B, D)
    out = _simple_cal_sc(h, r, t, rel, rel_inv_table)
    return out[:, None]


# speedup vs baseline: 1.4863x; 1.4863x over previous
import functools

import jax
import jax.numpy as jnp
from jax import lax
from jax.experimental import pallas as pl
from jax.experimental.pallas import tpu as pltpu
from jax.experimental.pallas import tpu_sc as plsc

B = 16384
D = 64
NC = 2
NS = 16
NW = NC * NS
N_PER_W = B // NW
CHUNK = 128
N_CHUNKS = N_PER_W // CHUNK
LANES = 16
GBATCH = 32


def _sc_body(h_hbm, r_hbm, t_hbm, rel_hbm, table_hbm, out_hbm,
             idx_v, rows_v, h_v, r_v, t_v, out_v, sem):
    wid = lax.axis_index("s") * NC + lax.axis_index("c")
    base = wid * N_PER_W
    for c in range(N_CHUNKS):
        row0 = base + c * CHUNK
        pltpu.sync_copy(rel_hbm.at[pl.ds(row0, CHUNK)], idx_v)
        pltpu.sync_copy(h_hbm.at[pl.ds(row0, CHUNK), :], h_v)
        pltpu.sync_copy(r_hbm.at[pl.ds(row0, CHUNK), :], r_v)
        pltpu.sync_copy(t_hbm.at[pl.ds(row0, CHUNK), :], t_v)

        lane = lax.iota(jnp.int32, LANES)

        def fire_body(g, carry):
            cps = []
            for j16 in range(GBATCH // LANES):
                ivec = idx_v[pl.ds(g * GBATCH + j16 * LANES, LANES)]
                for j in range(LANES):
                    row = g * GBATCH + j16 * LANES + j
                    r_idx = jnp.sum(jnp.where(lane == j, ivec, 0))
                    cps.append(pltpu.async_copy(
                        table_hbm.at[pl.ds(r_idx, 1), :],
                        rows_v.at[pl.ds(row, 1), :], sem))
            for cp in cps:
                cp.wait()
            return carry

        lax.fori_loop(0, CHUNK // GBATCH, fire_body, 0)

        def group_body(g, carry, c=c):
            rowvec = lax.iota(jnp.int32, LANES) + g * LANES
            acc = jnp.zeros((LANES,), jnp.float32)
            for d in range(D):
                dvec = jnp.full((LANES,), d, jnp.int32)
                hv = plsc.load_gather(h_v, [rowvec, dvec])
                tv = plsc.load_gather(t_v, [rowvec, dvec])
                rv = plsc.load_gather(r_v, [rowvec, dvec])
                gv = plsc.load_gather(rows_v, [rowvec, dvec])
                acc = acc + hv * tv * (rv + gv)
            out_v[pl.ds(c * CHUNK + g * LANES, LANES)] = acc * 0.5
            return carry

        lax.fori_loop(0, CHUNK // LANES, group_body, 0)
    pltpu.sync_copy(out_v, out_hbm.at[pl.ds(base, N_PER_W)])


@functools.partial(
    pl.kernel,
    out_type=jax.ShapeDtypeStruct((B,), jnp.float32),
    mesh=plsc.VectorSubcoreMesh(core_axis_name="c", subcore_axis_name="s"),
    compiler_params=pltpu.CompilerParams(needs_layout_passes=False),
    scratch_types=[
        pltpu.VMEM((CHUNK,), jnp.int32),
        pltpu.VMEM((CHUNK, D), jnp.float32),
        pltpu.VMEM((CHUNK, D), jnp.float32),
        pltpu.VMEM((CHUNK, D), jnp.float32),
        pltpu.VMEM((CHUNK, D), jnp.float32),
        pltpu.VMEM((N_PER_W,), jnp.float32),
        pltpu.SemaphoreType.DMA,
    ],
)
def _simple_cal_sc(h_hbm, r_hbm, t_hbm, rel_hbm, table_hbm, out_hbm,
                   idx_v, rows_v, h_v, r_v, t_v, out_v, sem):
    _sc_body(h_hbm, r_hbm, t_hbm, rel_hbm, table_hbm, out_hbm,
             idx_v, rows_v, h_v, r_v, t_v, out_v, sem)


def kernel(x0, x1, x2, rel, rel_inv_table):
    h = x0.reshape(B, D)
    r = x1.reshape(B, D)
    t = x2.reshape(B, D)
    out = _simple_cal_sc(h, r, t, rel, rel_inv_table)
    return out[:, None]
